# baseline (speedup 1.0000x reference)
import functools

import jax
import jax.numpy as jnp
from jax import lax
from jax.experimental import pallas as pl
from jax.experimental.pallas import tpu as pltpu
from jax.experimental.pallas import tpu_sc as plsc

SEQ = 2048
DIM = 2048
BATCH = 4
ROWS_PER_TILE = 1024
TOPK = SEQ // 2

_NC = 2
_NS = 16
_NW = _NC * _NS
_RPW = (BATCH * TOPK) // _NW
_CHUNK = 32
_NCHUNK = _RPW // _CHUNK


def _logits_body(x_ref, w_ref, wblk_ref, out_ref, wb_ref):
    xt = x_ref[...].astype(jnp.bfloat16).astype(jnp.float32)
    w = w_ref[...].astype(jnp.bfloat16).astype(jnp.float32)
    out_ref[...] = jnp.sum(xt * w, axis=1, keepdims=True)
    wb_ref[...] = wblk_ref[...].astype(jnp.bfloat16)


def _mask_body(lcol_ref, lrow_ref, mask_ref, *, k, rows, seq):
    s = pl.program_id(1)
    lc = lcol_ref[0]
    lr = lrow_ref[0]
    i_idx = lax.broadcasted_iota(jnp.int32, (rows, seq), 0) + s * rows
    j_idx = lax.broadcasted_iota(jnp.int32, (rows, seq), 1)
    beats = (lr > lc) | ((lr == lc) & (j_idx < i_idx))
    ones = jnp.ones((seq, 1), jnp.bfloat16)
    cnt = jnp.dot(beats.astype(jnp.bfloat16), ones,
                  preferred_element_type=jnp.float32)
    mask_ref[0] = (cnt < k).astype(jnp.float32)


def _gather_body(mask_hbm, x_hbm, xc_hbm, idx_hbm,
                 mask_v, idx_v, rows_v, sem):
    wid = lax.axis_index("s") * _NC + lax.axis_index("c")
    b = wid // 8
    q = wid % 8
    pltpu.sync_copy(mask_hbm.at[b], mask_v)

    lane_i = lax.broadcasted_iota(jnp.int32, (16,), 0)
    lane_f = lane_i.astype(jnp.float32)

    def body(i, ptr_f):
        mf = mask_v[pl.ds(i * 16, 16)]
        pre = plsc.cumsum(mf)
        vals = lane_i + (i * 16 + b * SEQ)
        tgt_f = mf * (ptr_f + pre - 1.0) + (1.0 - mf) * (TOPK + lane_f)
        plsc.store_scatter(idx_v, [tgt_f.astype(jnp.int32)], vals)
        return ptr_f + jnp.sum(mf)

    lax.fori_loop(0, SEQ // 16, body, jnp.float32(0.0))

    @pl.when(q == 0)
    def _():
        pltpu.sync_copy(idx_v.at[pl.ds(0, TOPK)],
                        idx_hbm.at[pl.ds(b * TOPK, TOPK)])

    for c in range(_NCHUNK):
        src = idx_v.at[pl.ds(q * _RPW + c * _CHUNK, _CHUNK)]
        pltpu.async_copy(x_hbm.at[src], rows_v, sem).wait()
        dst = wid * _RPW + c * _CHUNK
        pltpu.sync_copy(rows_v, xc_hbm.at[pl.ds(dst, _CHUNK)])


def _mm_body(xc_ref, w_ref, y_ref):
    y_ref[...] = lax.dot_general(xc_ref[...].astype(jnp.bfloat16), w_ref[...],
                                 (((1,), (1,)), ((), ())),
                                 preferred_element_type=jnp.float32)


def _scatter_body(idx_ref, y_ref, xin_ref, out_ref):
    del idx_ref, xin_ref
    out_ref[...] = y_ref[...]


def kernel(x, W_block, W_router):
    B, S, D = x.shape
    k = int(S * 0.5)
    xf = x.reshape(B * S, D)
    rows = ROWS_PER_TILE
    n_tiles = (B * S) // rows

    wrows = D // n_tiles
    logits, wb = pl.pallas_call(
        _logits_body,
        grid=(n_tiles,),
        in_specs=[
            pl.BlockSpec((rows, D), lambda i: (i, 0)),
            pl.BlockSpec((1, D), lambda i: (0, 0)),
            pl.BlockSpec((wrows, D), lambda i: (i, 0)),
        ],
        out_specs=[
            pl.BlockSpec((rows, 1), lambda i: (i, 0)),
            pl.BlockSpec((wrows, D), lambda i: (i, 0)),
        ],
        out_shape=[
            jax.ShapeDtypeStruct((B * S, 1), jnp.float32),
            jax.ShapeDtypeStruct((D, D), jnp.bfloat16),
        ],
    )(xf, W_router, W_block)

    mrows = 256
    mask = pl.pallas_call(
        functools.partial(_mask_body, k=k, rows=mrows, seq=S),
        grid=(B, S // mrows),
        in_specs=[
            pl.BlockSpec((1, mrows, 1), lambda b, s: (b, s, 0)),
            pl.BlockSpec((1, 1, S), lambda b, s: (b, 0, 0)),
        ],
        out_specs=pl.BlockSpec((1, mrows, 1), lambda b, s: (b, s, 0)),
        out_shape=jax.ShapeDtypeStruct((B, S, 1), jnp.float32),
    )(logits.reshape(B, S, 1), logits.reshape(B, 1, S))

    mesh = plsc.VectorSubcoreMesh(core_axis_name="c", subcore_axis_name="s")
    xc, idx = pl.kernel(
        _gather_body,
        mesh=mesh,
        out_type=[
            jax.ShapeDtypeStruct((B * k, D), jnp.float32),
            jax.ShapeDtypeStruct((B * k,), jnp.int32),
        ],
        scratch_types=[
            pltpu.VMEM((S,), jnp.float32),
            pltpu.VMEM((TOPK + 16,), jnp.int32),
            pltpu.VMEM((_CHUNK, D), jnp.float32),
            pltpu.SemaphoreType.DMA,
        ],
        compiler_params=pltpu.CompilerParams(needs_layout_passes=False),
    )(mask.reshape(B, S), xf)

    y = pl.pallas_call(
        _mm_body,
        grid=((B * k) // rows,),
        in_specs=[
            pl.BlockSpec((rows, D), lambda i: (i, 0)),
            pl.BlockSpec((D, D), lambda i: (0, 0)),
        ],
        out_specs=pl.BlockSpec((rows, D), lambda i: (i, 0)),
        out_shape=jax.ShapeDtypeStruct((B * k, D), jnp.float32),
    )(xc, wb)

    grid_spec = pltpu.PrefetchScalarGridSpec(
        num_scalar_prefetch=1,
        grid=(B * k,),
        in_specs=[
            pl.BlockSpec((1, 1, D), lambda i, idx_ref: (i, 0, 0)),
            pl.BlockSpec(memory_space=pl.ANY),
        ],
        out_specs=pl.BlockSpec((1, 1, D), lambda i, idx_ref: (idx_ref[i], 0, 0)),
    )
    out = pl.pallas_call(
        _scatter_body,
        grid_spec=grid_spec,
        out_shape=jax.ShapeDtypeStruct((B * S, 1, D), jnp.float32),
        input_output_aliases={2: 0},
    )(idx, y.reshape(B * k, 1, D), xf.reshape(B * S, 1, D))

    return out.reshape(B, S, D)


# speedup vs baseline: 16.2084x; 16.2084x over previous
import functools

import jax
import jax.numpy as jnp
from jax import lax
from jax.experimental import pallas as pl
from jax.experimental.pallas import tpu as pltpu

SEQ = 2048
DIM = 2048
BATCH = 4
ROWS_PER_TILE = 1024


def _logits_body(x_ref, w_ref, wblk_ref, out_ref, wb_ref):
    xt = x_ref[...].astype(jnp.bfloat16).astype(jnp.float32)
    w = w_ref[...].astype(jnp.bfloat16).astype(jnp.float32)
    out_ref[...] = jnp.sum(xt * w, axis=1, keepdims=True)
    wb_ref[...] = wblk_ref[...].astype(jnp.bfloat16)


def _block_body(x_ref, w_ref, lcol_ref, lrow_ref, out_ref, *, k, rows, seq):
    s = pl.program_id(1)
    xt = x_ref[0]
    acc = lax.dot_general(xt.astype(jnp.bfloat16), w_ref[...],
                          (((1,), (1,)), ((), ())),
                          preferred_element_type=jnp.float32)
    lc = lcol_ref[0]
    lr = lrow_ref[0]
    i_idx = lax.broadcasted_iota(jnp.int32, (rows, seq), 0) + s * rows
    j_idx = lax.broadcasted_iota(jnp.int32, (rows, seq), 1)
    beats = (lr > lc) | ((lr == lc) & (j_idx < i_idx))
    ones = jnp.ones((seq, 1), jnp.bfloat16)
    cnt = jnp.dot(beats.astype(jnp.bfloat16), ones,
                  preferred_element_type=jnp.float32)
    out_ref[0] = jnp.where(cnt < k, acc, xt)


def kernel(x, W_block, W_router):
    B, S, D = x.shape
    k = int(S * 0.5)
    xf = x.reshape(B * S, D)
    rows = ROWS_PER_TILE
    n_tiles = (B * S) // rows

    wrows = D // n_tiles
    logits, wb = pl.pallas_call(
        _logits_body,
        grid=(n_tiles,),
        in_specs=[
            pl.BlockSpec((rows, D), lambda i: (i, 0)),
            pl.BlockSpec((1, D), lambda i: (0, 0)),
            pl.BlockSpec((wrows, D), lambda i: (i, 0)),
        ],
        out_specs=[
            pl.BlockSpec((rows, 1), lambda i: (i, 0)),
            pl.BlockSpec((wrows, D), lambda i: (i, 0)),
        ],
        out_shape=[
            jax.ShapeDtypeStruct((B * S, 1), jnp.float32),
            jax.ShapeDtypeStruct((D, D), jnp.bfloat16),
        ],
    )(xf, W_router, W_block)
    out = pl.pallas_call(
        functools.partial(_block_body, k=k, rows=rows, seq=S),
        grid=(B, S // rows),
        in_specs=[
            pl.BlockSpec((1, rows, D), lambda b, s: (b, s, 0)),
            pl.BlockSpec((D, D), lambda b, s: (0, 0)),
            pl.BlockSpec((1, rows, 1), lambda b, s: (b, s, 0)),
            pl.BlockSpec((1, 1, S), lambda b, s: (b, 0, 0)),
        ],
        out_specs=pl.BlockSpec((1, rows, D), lambda b, s: (b, s, 0)),
        out_shape=jax.ShapeDtypeStruct((B, S, D), jnp.float32),
        compiler_params=pltpu.CompilerParams(
            vmem_limit_bytes=100 * 1024 * 1024),
    )(x, wb, logits.reshape(B, S, 1), logits.reshape(B, 1, S))

    return out
